# Initial kernel scaffold; baseline (speedup 1.0000x reference)
import jax
import jax.numpy as jnp
from jax.experimental import pallas as pl


def kernel(A_edge_index, A_values, H, W, bias):
    raise NotImplementedError("write your pallas kernel here")


# speedup vs baseline: 4.5054x; 4.5054x over previous
import functools

import jax
import jax.numpy as jnp
from jax import lax
from jax.experimental import pallas as pl
from jax.experimental.pallas import tpu as pltpu
from jax.experimental.pallas import tpu_sc as plsc

N_NODES = 10000
N_EDGES = 320000
D_IN = 128
D_OUT = 128

NC = 2
NS = 16
NW = NC * NS
EDGES_PER_TILE = N_EDGES // NW
CHUNK = 80
NCHUNKS = EDGES_PER_TILE // CHUNK
N_PAD = 10240
ROWS_PER_TILE = N_PAD // NS
LANES = 16


@functools.partial(
    pl.kernel,
    out_type=jax.ShapeDtypeStruct((NC, N_PAD, D_IN), jnp.float32),
    mesh=plsc.VectorSubcoreMesh(core_axis_name="c", subcore_axis_name="s"),
    scratch_types=[
        pltpu.VMEM((CHUNK,), jnp.int32),
        pltpu.VMEM((CHUNK,), jnp.int32),
        pltpu.VMEM((CHUNK,), jnp.float32),
        pltpu.VMEM((CHUNK, D_IN), jnp.float32),
        pltpu.VMEM_SHARED((N_PAD, D_IN), jnp.float32),
        pltpu.SemaphoreType.DMA,
    ],
)
def _sc_spmm(h_hbm, col_hbm, row_hbm, val_hbm, zeros_hbm, out_hbm,
             col_v, row_v, val_v, rows_v, acc, sem):
    cid = lax.axis_index("c")
    sid = lax.axis_index("s")
    wid = sid * NC + cid

    pltpu.sync_copy(zeros_hbm, acc.at[pl.ds(sid * ROWS_PER_TILE, ROWS_PER_TILE)])
    plsc.subcore_barrier()

    base0 = wid * EDGES_PER_TILE

    def chunk_body(c, carry):
        base = base0 + c * CHUNK
        pltpu.sync_copy(col_hbm.at[pl.ds(base, CHUNK)], col_v)
        pltpu.sync_copy(row_hbm.at[pl.ds(base, CHUNK)], row_v)
        pltpu.sync_copy(val_hbm.at[pl.ds(base, CHUNK)], val_v)
        pltpu.async_copy(h_hbm.at[col_v], rows_v, sem).wait()

        def group_body(g, carry2):
            v16 = val_v[pl.ds(g * LANES, LANES)]
            for l in range(LANES):
                vv = jnp.full((LANES,), v16[l], jnp.float32)
                e = g * LANES + l
                for j in range(D_IN // LANES):
                    sl = pl.ds(j * LANES, LANES)
                    rows_v[e, sl] = rows_v[e, sl] * vv
            return carry2

        lax.fori_loop(0, CHUNK // LANES, group_body, 0)
        pltpu.sync_copy(rows_v, acc.at[row_v], add=True)
        return carry

    lax.fori_loop(0, NCHUNKS, chunk_body, 0)
    plsc.subcore_barrier()
    pltpu.sync_copy(acc.at[pl.ds(sid * ROWS_PER_TILE, ROWS_PER_TILE)],
                    out_hbm.at[cid, pl.ds(sid * ROWS_PER_TILE, ROWS_PER_TILE)])


_BM = 1000


def _tc_body(p_ref, w_ref, b_ref, o_ref):
    s = p_ref[0] + p_ref[1]
    acc = jnp.dot(s, w_ref[...], preferred_element_type=jnp.float32)
    o_ref[...] = jnp.maximum(acc + b_ref[...], 0.0)


def _tc_combine(partials, W, bias2d):
    return pl.pallas_call(
        _tc_body,
        grid=(N_NODES // _BM,),
        in_specs=[
            pl.BlockSpec((NC, _BM, D_IN), lambda i: (0, i, 0)),
            pl.BlockSpec((D_IN, D_OUT), lambda i: (0, 0)),
            pl.BlockSpec((1, D_OUT), lambda i: (0, 0)),
        ],
        out_specs=pl.BlockSpec((_BM, D_OUT), lambda i: (i, 0)),
        out_shape=jax.ShapeDtypeStruct((N_NODES, D_OUT), jnp.float32),
    )(partials, W, bias2d)


def kernel(A_edge_index, A_values, H, W, bias):
    row = A_edge_index[0]
    col = A_edge_index[1]
    zeros = jnp.zeros((ROWS_PER_TILE, D_IN), jnp.float32)
    partials = _sc_spmm(H, col, row, A_values, zeros)
    return _tc_combine(partials, W, bias.reshape(1, D_OUT))


# speedup vs baseline: 7.2588x; 1.6111x over previous
import functools

import jax
import jax.numpy as jnp
from jax import lax
from jax.experimental import pallas as pl
from jax.experimental.pallas import tpu as pltpu
from jax.experimental.pallas import tpu_sc as plsc

N_NODES = 10000
N_EDGES = 320000
D_IN = 128
D_OUT = 128

NC = 2
NS = 16
NW = NC * NS
EDGES_PER_TILE = N_EDGES // NW
CHUNK = 80
NCHUNKS = EDGES_PER_TILE // CHUNK
PK = 2 * CHUNK
N_PAD = 10240
ROWS_PER_TILE = N_PAD // NS
LANES = 16
NGROUPS = CHUNK // LANES


@functools.partial(
    pl.kernel,
    out_type=jax.ShapeDtypeStruct((NC, N_PAD, D_IN), jnp.float32),
    mesh=plsc.VectorSubcoreMesh(core_axis_name="c", subcore_axis_name="s"),
    scratch_types=[
        pltpu.VMEM((PK,), jnp.int32),
        pltpu.VMEM((PK,), jnp.int32),
        pltpu.VMEM((CHUNK,), jnp.float32),
        pltpu.VMEM((CHUNK,), jnp.float32),
        pltpu.VMEM((CHUNK,), jnp.int32),
        pltpu.VMEM((CHUNK,), jnp.int32),
        pltpu.VMEM((CHUNK, D_IN), jnp.float32),
        pltpu.VMEM((CHUNK, D_IN), jnp.float32),
        pltpu.VMEM_SHARED((N_PAD, D_IN), jnp.float32),
        pltpu.SemaphoreType.DMA,
        pltpu.SemaphoreType.DMA,
        pltpu.SemaphoreType.DMA,
        pltpu.SemaphoreType.DMA,
    ],
)
def _sc_spmm(h_hbm, packed_hbm, val_hbm, out_hbm,
             p0, p1, v0, v1, r0, r1, b0, b1, acc, g0, g1, s0, s1):
    cid = lax.axis_index("c")
    sid = lax.axis_index("s")
    wid = sid * NC + cid
    base0 = wid * NCHUNKS

    zv = jnp.zeros((LANES,), jnp.float32)
    for e in range(CHUNK):
        for j in range(D_IN // LANES):
            b0[e, pl.ds(j * LANES, LANES)] = zv

    def zero_body(i, carry):
        pltpu.sync_copy(
            b0, acc.at[pl.ds(sid * ROWS_PER_TILE + i * CHUNK, CHUNK)])
        return carry

    lax.fori_loop(0, ROWS_PER_TILE // CHUNK, zero_body, 0)
    plsc.subcore_barrier()

    def load_packed(c, pv, vv_ref):
        pltpu.sync_copy(packed_hbm.at[pl.ds((base0 + c) * PK, PK)], pv)
        pltpu.sync_copy(
            val_hbm.at[pl.ds(base0 * CHUNK + c * CHUNK, CHUNK)], vv_ref)

    def copy_row_idx(pv, rv):
        for i in range(NGROUPS):
            rv[pl.ds(i * LANES, LANES)] = pv[pl.ds(CHUNK + i * LANES, LANES)]

    def start_gather(pv, bv, sem):
        pltpu.async_copy(h_hbm.at[pv.at[pl.ds(0, CHUNK)]], bv, sem)

    def wait_gather(pv, bv, sem):
        pltpu.make_async_copy(h_hbm.at[pv.at[pl.ds(0, CHUNK)]], bv, sem).wait()

    def scale(bv, vv_ref):
        def group_body(g, carry):
            v16 = vv_ref[pl.ds(g * LANES, LANES)]
            for l in range(LANES):
                vv = jnp.full((LANES,), v16[l], jnp.float32)
                e = g * LANES + l
                for j in range(D_IN // LANES):
                    sl = pl.ds(j * LANES, LANES)
                    bv[e, sl] = bv[e, sl] * vv
            return carry

        lax.fori_loop(0, NGROUPS, group_body, 0)

    def start_scatter(bv, rv, sem):
        pltpu.async_copy(bv, acc.at[rv], sem, add=True)

    def wait_scatter(bv, rv, sem):
        pltpu.make_async_copy(bv, acc.at[rv], sem).wait()

    def step(pv, vv, rv, bv, gsem, ssem, qv, qvv, qr, qb, qg, qs, c_next):
        wait_scatter(qb, qr, qs)
        load_packed(c_next, qv, qvv)
        copy_row_idx(qv, qr)
        start_gather(qv, qb, qg)
        wait_gather(pv, bv, gsem)
        scale(bv, vv)
        start_scatter(bv, rv, ssem)

    load_packed(0, p0, v0)
    copy_row_idx(p0, r0)
    start_gather(p0, b0, g0)
    load_packed(1, p1, v1)
    copy_row_idx(p1, r1)
    start_gather(p1, b1, g1)
    wait_gather(p0, b0, g0)
    scale(b0, v0)
    start_scatter(b0, r0, s0)

    def pair_body(k, carry):
        step(p1, v1, r1, b1, g1, s1, p0, v0, r0, b0, g0, s0, 2 * k + 2)
        step(p0, v0, r0, b0, g0, s0, p1, v1, r1, b1, g1, s1, 2 * k + 3)
        return carry

    lax.fori_loop(0, (NCHUNKS - 3) // 2, pair_body, 0)

    step(p1, v1, r1, b1, g1, s1, p0, v0, r0, b0, g0, s0, NCHUNKS - 1)
    wait_scatter(b1, r1, s1)
    wait_gather(p0, b0, g0)
    scale(b0, v0)
    start_scatter(b0, r0, s0)
    wait_scatter(b0, r0, s0)

    plsc.subcore_barrier()
    pltpu.sync_copy(acc.at[pl.ds(sid * ROWS_PER_TILE, ROWS_PER_TILE)],
                    out_hbm.at[cid, pl.ds(sid * ROWS_PER_TILE, ROWS_PER_TILE)])


_BM = 1000


def _tc_body(p_ref, w_ref, b_ref, o_ref):
    s = p_ref[0] + p_ref[1]
    acc = jnp.dot(s, w_ref[...], preferred_element_type=jnp.float32)
    o_ref[...] = jnp.maximum(acc + b_ref[...], 0.0)


def _tc_combine(partials, W, bias2d):
    return pl.pallas_call(
        _tc_body,
        grid=(N_NODES // _BM,),
        in_specs=[
            pl.BlockSpec((NC, _BM, D_IN), lambda i: (0, i, 0)),
            pl.BlockSpec((D_IN, D_OUT), lambda i: (0, 0)),
            pl.BlockSpec((1, D_OUT), lambda i: (0, 0)),
        ],
        out_specs=pl.BlockSpec((_BM, D_OUT), lambda i: (i, 0)),
        out_shape=jax.ShapeDtypeStruct((N_NODES, D_OUT), jnp.float32),
    )(partials, W, bias2d)


def kernel(A_edge_index, A_values, H, W, bias):
    row = A_edge_index[0]
    col = A_edge_index[1]
    nchunks_total = N_EDGES // CHUNK
    packed = jnp.concatenate(
        [col.reshape(nchunks_total, CHUNK),
         row.reshape(nchunks_total, CHUNK)], axis=1).reshape(-1)
    partials = _sc_spmm(H, packed, A_values)
    return _tc_combine(partials, W, bias.reshape(1, D_OUT))
